# Initial kernel scaffold; baseline (speedup 1.0000x reference)
import jax
import jax.numpy as jnp
from jax.experimental import pallas as pl


def kernel(x, edge_index, W_self1, W_neigh1, b1, W_self2, W_neigh2, b2, W_cls, b_cls):
    raise NotImplementedError("write your pallas kernel here")


# speedup vs baseline: 4.9694x; 4.9694x over previous
import functools

import jax
import jax.numpy as jnp
from jax import lax
from jax.experimental import pallas as pl
from jax.experimental.pallas import tpu as pltpu
from jax.experimental.pallas import tpu_sc as plsc

N = 10000
E = 320000
D = 128
OUT = 64

NP = 10240
NC = 2
NS = 16
NW = NC * NS
EPW = E // NW
K = 80
NCHUNK = EPW // K
RPT = NP // NS

_SC_MESH = plsc.VectorSubcoreMesh(core_axis_name="c", subcore_axis_name="s")


@functools.partial(
    pl.kernel,
    out_type=[
        jax.ShapeDtypeStruct((NC, NP, D), jnp.float32),
        jax.ShapeDtypeStruct((NC, NP), jnp.float32),
    ],
    mesh=_SC_MESH,
    scratch_types=[
        pltpu.VMEM((K,), jnp.int32),
        pltpu.VMEM((K,), jnp.int32),
        pltpu.VMEM((K, D), jnp.float32),
        pltpu.VMEM((128, D), jnp.float32),
        pltpu.VMEM((RPT,), jnp.float32),
        pltpu.VMEM((K,), jnp.float32),
        pltpu.VMEM_SHARED((NP, D), jnp.float32),
        pltpu.VMEM_SHARED((NP,), jnp.float32),
        pltpu.SemaphoreType.DMA,
    ],
)
def _sc_agg(x_hbm, src_hbm, dst_hbm, agg_out, deg_out,
            src_v, dst_v, rows_v, zeros_v, stage1_v, ones_v, acc_sh, deg_sh,
            sem):
    c = lax.axis_index("c")
    s = lax.axis_index("s")
    wid = c * NS + s
    zero16 = jnp.zeros((16,), jnp.float32)
    one16 = jnp.ones((16,), jnp.float32)

    def _zero_zv(i, carry):
        zeros_v[i // 8, pl.ds((i % 8) * 16, 16)] = zero16
        return carry

    lax.fori_loop(0, 128 * (D // 16), _zero_zv, 0)

    def _zero_s1(i, carry):
        stage1_v[pl.ds(i * 16, 16)] = zero16
        return carry

    lax.fori_loop(0, RPT // 16, _zero_s1, 0)

    for i in range(K // 16):
        ones_v[pl.ds(i * 16, 16)] = one16

    tb = s * RPT
    for j in range(RPT // 128):
        pltpu.sync_copy(zeros_v, acc_sh.at[pl.ds(tb + j * 128, 128)])
    pltpu.sync_copy(stage1_v, deg_sh.at[pl.ds(tb, RPT)])
    plsc.subcore_barrier()

    ebase = wid * EPW

    def _chunk(ci, carry):
        off = pl.multiple_of(ebase + ci * K, 8)
        pltpu.sync_copy(src_hbm.at[pl.ds(off, K)], src_v)
        pltpu.sync_copy(dst_hbm.at[pl.ds(off, K)], dst_v)
        pltpu.async_copy(x_hbm.at[src_v], rows_v, sem).wait()
        pltpu.sync_copy(rows_v, acc_sh.at[dst_v], add=True)
        pltpu.sync_copy(ones_v, deg_sh.at[dst_v], add=True)
        return carry

    lax.fori_loop(0, NCHUNK, _chunk, 0)
    plsc.subcore_barrier()

    for j in range(RPT // K):
        r0 = tb + j * K
        pltpu.sync_copy(acc_sh.at[pl.ds(r0, K)], rows_v)
        pltpu.sync_copy(rows_v, agg_out.at[c, pl.ds(r0, K)])
    pltpu.sync_copy(deg_sh.at[pl.ds(tb, RPT)], stage1_v)
    pltpu.sync_copy(stage1_v, deg_out.at[c, pl.ds(tb, RPT)])


_BLK = 2000


def _degscale_body(deg_ref, scale_ref):
    deg = jnp.sum(deg_ref[...], axis=0)
    scale_ref[...] = (1.0 / jnp.maximum(deg, 1.0))[:, None]


_tc_degscale = pl.pallas_call(
    _degscale_body,
    out_shape=jax.ShapeDtypeStruct((NP, 1), jnp.float32),
)


def _layer1_body(x_ref, agg_ref, scale_ref, ws_ref, wn_ref, b_ref, out_ref):
    aggs = agg_ref[0] + agg_ref[1]
    hn = aggs * scale_ref[...]
    h = (jnp.dot(x_ref[...], ws_ref[...], preferred_element_type=jnp.float32)
         + jnp.dot(hn, wn_ref[...], preferred_element_type=jnp.float32)
         + b_ref[...])
    out_ref[...] = jnp.maximum(h, 0.0)


_tc_layer1 = pl.pallas_call(
    _layer1_body,
    grid=(N // _BLK,),
    in_specs=[
        pl.BlockSpec((_BLK, D), lambda i: (i, 0)),
        pl.BlockSpec((NC, _BLK, D), lambda i: (0, i, 0)),
        pl.BlockSpec((_BLK, 1), lambda i: (i, 0)),
        pl.BlockSpec((D, D), lambda i: (0, 0)),
        pl.BlockSpec((D, D), lambda i: (0, 0)),
        pl.BlockSpec((1, D), lambda i: (0, 0)),
    ],
    out_specs=pl.BlockSpec((_BLK, D), lambda i: (i, 0)),
    out_shape=jax.ShapeDtypeStruct((N, D), jnp.float32),
)


def _layer2_body(h_ref, agg_ref, scale_ref, ws_ref, wn_ref, b_ref, wc_ref,
                 bc_ref, h2_ref, out_ref):
    aggs = agg_ref[0] + agg_ref[1]
    hn = aggs * scale_ref[...]
    h = (jnp.dot(h_ref[...], ws_ref[...], preferred_element_type=jnp.float32)
         + jnp.dot(hn, wn_ref[...], preferred_element_type=jnp.float32)
         + b_ref[...])
    h2 = jnp.maximum(h, 0.0)
    h2_ref[...] = h2
    out_ref[...] = (jnp.dot(h2, wc_ref[...], preferred_element_type=jnp.float32)
                    + bc_ref[...])


_tc_layer2 = pl.pallas_call(
    _layer2_body,
    grid=(N // _BLK,),
    in_specs=[
        pl.BlockSpec((_BLK, D), lambda i: (i, 0)),
        pl.BlockSpec((NC, _BLK, D), lambda i: (0, i, 0)),
        pl.BlockSpec((_BLK, 1), lambda i: (i, 0)),
        pl.BlockSpec((D, D), lambda i: (0, 0)),
        pl.BlockSpec((D, D), lambda i: (0, 0)),
        pl.BlockSpec((1, D), lambda i: (0, 0)),
        pl.BlockSpec((D, OUT), lambda i: (0, 0)),
        pl.BlockSpec((1, OUT), lambda i: (0, 0)),
    ],
    out_specs=[
        pl.BlockSpec((_BLK, D), lambda i: (i, 0)),
        pl.BlockSpec((_BLK, OUT), lambda i: (i, 0)),
    ],
    out_shape=[
        jax.ShapeDtypeStruct((N, D), jnp.float32),
        jax.ShapeDtypeStruct((N, OUT), jnp.float32),
    ],
)


def kernel(x, edge_index, W_self1, W_neigh1, b1, W_self2, W_neigh2, b2,
           W_cls, b_cls):
    src = edge_index[0]
    dst = edge_index[1]
    agg1, deg = _sc_agg(x, src, dst)
    scale = _tc_degscale(deg)
    h1 = _tc_layer1(x, agg1, scale, W_self1, W_neigh1, b1.reshape(1, D))
    agg2, _ = _sc_agg(h1, src, dst)
    h2, logits = _tc_layer2(h1, agg2, scale, W_self2, W_neigh2,
                            b2.reshape(1, D), W_cls, b_cls.reshape(1, OUT))
    return (logits, h2)


# speedup vs baseline: 9.2157x; 1.8545x over previous
import functools

import jax
import jax.numpy as jnp
from jax import lax
from jax.experimental import pallas as pl
from jax.experimental.pallas import tpu as pltpu
from jax.experimental.pallas import tpu_sc as plsc

N = 10000
E = 320000
D = 128
OUT = 64

NP = 10240
NC = 2
NS = 16
NW = NC * NS
EPW = E // NW
K = 80
NCHUNK = EPW // K
RPT = NP // NS

_SC_MESH = plsc.VectorSubcoreMesh(core_axis_name="c", subcore_axis_name="s")


@functools.partial(
    pl.kernel,
    out_type=[
        jax.ShapeDtypeStruct((NC, NP, D), jnp.float32),
        jax.ShapeDtypeStruct((NC, NP), jnp.float32),
    ],
    mesh=_SC_MESH,
    scratch_types=[
        pltpu.VMEM((EPW,), jnp.int32),
        pltpu.VMEM((EPW,), jnp.int32),
        pltpu.VMEM((K,), jnp.int32),
        pltpu.VMEM((K,), jnp.int32),
        pltpu.VMEM((K,), jnp.int32),
        pltpu.VMEM((K,), jnp.int32),
        pltpu.VMEM((K, D), jnp.float32),
        pltpu.VMEM((K, D), jnp.float32),
        pltpu.VMEM((RPT,), jnp.float32),
        pltpu.VMEM((K,), jnp.float32),
        pltpu.VMEM_SHARED((NP, D), jnp.float32),
        pltpu.VMEM_SHARED((NP,), jnp.float32),
        pltpu.SemaphoreType.DMA,
        pltpu.SemaphoreType.DMA,
    ],
)
def _sc_agg(x_hbm, src_hbm, dst_hbm, agg_out, deg_out,
            src_all, dst_all, srcA, srcB, dstA, dstB, rowsA, rowsB,
            stage1_v, ones_v, acc_sh, deg_sh, semA, semB):
    c = lax.axis_index("c")
    s = lax.axis_index("s")
    wid = c * NS + s
    zero16 = jnp.zeros((16,), jnp.float32)
    one16 = jnp.ones((16,), jnp.float32)

    def _zero_rows(i, carry):
        rowsA[i // 8, pl.ds((i % 8) * 16, 16)] = zero16
        return carry

    lax.fori_loop(0, K * (D // 16), _zero_rows, 0)

    def _zero_s1(i, carry):
        stage1_v[pl.ds(i * 16, 16)] = zero16
        return carry

    lax.fori_loop(0, RPT // 16, _zero_s1, 0)

    for i in range(K // 16):
        ones_v[pl.ds(i * 16, 16)] = one16

    tb = s * RPT
    ebase = pl.multiple_of(wid * EPW, 8)
    pltpu.async_copy(src_hbm.at[pl.ds(ebase, EPW)], src_all, semA)
    pltpu.async_copy(dst_hbm.at[pl.ds(ebase, EPW)], dst_all, semB)
    for j in range(RPT // K):
        pltpu.sync_copy(rowsA, acc_sh.at[pl.ds(tb + j * K, K)])
    pltpu.sync_copy(stage1_v, deg_sh.at[pl.ds(tb, RPT)])
    pltpu.make_async_copy(src_hbm.at[pl.ds(ebase, EPW)], src_all, semA).wait()
    pltpu.make_async_copy(dst_hbm.at[pl.ds(ebase, EPW)], dst_all, semB).wait()
    plsc.subcore_barrier()

    def _stage(lo, sbuf, dbuf):
        for i in range(K // 16):
            sbuf[pl.ds(i * 16, 16)] = src_all[pl.ds(lo + i * 16, 16)]
            dbuf[pl.ds(i * 16, 16)] = dst_all[pl.ds(lo + i * 16, 16)]

    def _scatter(rows, dbuf):
        pltpu.sync_copy(rows, acc_sh.at[dbuf], add=True)
        pltpu.sync_copy(ones_v, deg_sh.at[dbuf], add=True)

    _stage(0, srcA, dstA)
    pltpu.async_copy(x_hbm.at[srcA], rowsA, semA)

    def _pair(j, carry):
        lo0 = 2 * j * K
        pltpu.make_async_copy(x_hbm.at[srcA], rowsA, semA).wait()
        _stage(lo0 + K, srcB, dstB)
        pltpu.async_copy(x_hbm.at[srcB], rowsB, semB)
        _scatter(rowsA, dstA)
        pltpu.make_async_copy(x_hbm.at[srcB], rowsB, semB).wait()
        _stage(lo0 + 2 * K, srcA, dstA)
        pltpu.async_copy(x_hbm.at[srcA], rowsA, semA)
        _scatter(rowsB, dstB)
        return carry

    lax.fori_loop(0, (NCHUNK - 1) // 2, _pair, 0)
    pltpu.make_async_copy(x_hbm.at[srcA], rowsA, semA).wait()
    _scatter(rowsA, dstA)
    plsc.subcore_barrier()

    for j in range(RPT // K):
        r0 = tb + j * K
        pltpu.sync_copy(acc_sh.at[pl.ds(r0, K)], rowsA)
        pltpu.sync_copy(rowsA, agg_out.at[c, pl.ds(r0, K)])
    pltpu.sync_copy(deg_sh.at[pl.ds(tb, RPT)], stage1_v)
    pltpu.sync_copy(stage1_v, deg_out.at[c, pl.ds(tb, RPT)])


_BLK = 2000


def _degscale_body(deg_ref, scale_ref):
    deg = jnp.sum(deg_ref[...], axis=0)
    scale_ref[...] = (1.0 / jnp.maximum(deg, 1.0))[:, None]


_tc_degscale = pl.pallas_call(
    _degscale_body,
    out_shape=jax.ShapeDtypeStruct((NP, 1), jnp.float32),
)


def _layer1_body(x_ref, agg_ref, scale_ref, ws_ref, wn_ref, b_ref, out_ref):
    aggs = agg_ref[0] + agg_ref[1]
    hn = aggs * scale_ref[...]
    h = (jnp.dot(x_ref[...], ws_ref[...], preferred_element_type=jnp.float32)
         + jnp.dot(hn, wn_ref[...], preferred_element_type=jnp.float32)
         + b_ref[...])
    out_ref[...] = jnp.maximum(h, 0.0)


_tc_layer1 = pl.pallas_call(
    _layer1_body,
    grid=(N // _BLK,),
    in_specs=[
        pl.BlockSpec((_BLK, D), lambda i: (i, 0)),
        pl.BlockSpec((NC, _BLK, D), lambda i: (0, i, 0)),
        pl.BlockSpec((_BLK, 1), lambda i: (i, 0)),
        pl.BlockSpec((D, D), lambda i: (0, 0)),
        pl.BlockSpec((D, D), lambda i: (0, 0)),
        pl.BlockSpec((1, D), lambda i: (0, 0)),
    ],
    out_specs=pl.BlockSpec((_BLK, D), lambda i: (i, 0)),
    out_shape=jax.ShapeDtypeStruct((N, D), jnp.float32),
)


def _layer2_body(h_ref, agg_ref, scale_ref, ws_ref, wn_ref, b_ref, wc_ref,
                 bc_ref, h2_ref, out_ref):
    aggs = agg_ref[0] + agg_ref[1]
    hn = aggs * scale_ref[...]
    h = (jnp.dot(h_ref[...], ws_ref[...], preferred_element_type=jnp.float32)
         + jnp.dot(hn, wn_ref[...], preferred_element_type=jnp.float32)
         + b_ref[...])
    h2 = jnp.maximum(h, 0.0)
    h2_ref[...] = h2
    out_ref[...] = (jnp.dot(h2, wc_ref[...], preferred_element_type=jnp.float32)
                    + bc_ref[...])


_tc_layer2 = pl.pallas_call(
    _layer2_body,
    grid=(N // _BLK,),
    in_specs=[
        pl.BlockSpec((_BLK, D), lambda i: (i, 0)),
        pl.BlockSpec((NC, _BLK, D), lambda i: (0, i, 0)),
        pl.BlockSpec((_BLK, 1), lambda i: (i, 0)),
        pl.BlockSpec((D, D), lambda i: (0, 0)),
        pl.BlockSpec((D, D), lambda i: (0, 0)),
        pl.BlockSpec((1, D), lambda i: (0, 0)),
        pl.BlockSpec((D, OUT), lambda i: (0, 0)),
        pl.BlockSpec((1, OUT), lambda i: (0, 0)),
    ],
    out_specs=[
        pl.BlockSpec((_BLK, D), lambda i: (i, 0)),
        pl.BlockSpec((_BLK, OUT), lambda i: (i, 0)),
    ],
    out_shape=[
        jax.ShapeDtypeStruct((N, D), jnp.float32),
        jax.ShapeDtypeStruct((N, OUT), jnp.float32),
    ],
)


def kernel(x, edge_index, W_self1, W_neigh1, b1, W_self2, W_neigh2, b2,
           W_cls, b_cls):
    src = edge_index[0]
    dst = edge_index[1]
    agg1, deg = _sc_agg(x, src, dst)
    scale = _tc_degscale(deg)
    h1 = _tc_layer1(x, agg1, scale, W_self1, W_neigh1, b1.reshape(1, D))
    agg2, _ = _sc_agg(h1, src, dst)
    h2, logits = _tc_layer2(h1, agg2, scale, W_self2, W_neigh2,
                            b2.reshape(1, D), W_cls, b_cls.reshape(1, OUT))
    return (logits, h2)
